# baseline (speedup 1.0000x reference)
import functools

import jax
import jax.numpy as jnp
from jax import lax
from jax.experimental import pallas as pl
from jax.experimental.pallas import tpu as pltpu
from jax.experimental.pallas import tpu_sc as plsc

N_NODES = 10000
D = 128
ALPHA = 0.5

NUM_CORES = 2
NUM_TILES = 16
CHUNK = 128
CHUNKS_PER_TILE = 158
EDGES_PER_TILE = CHUNKS_PER_TILE * CHUNK
E_PAD = EDGES_PER_TILE * NUM_TILES
ACC_ROWS = 10112
ROWS_PER_TILE = ACC_ROWS // NUM_TILES
X_PAD_ROWS = 10016


def _sc_aggregate(x_pad, src_pad, dst_pad, z2d, z1d):
    mesh = plsc.VectorSubcoreMesh(core_axis_name="c", subcore_axis_name="s")
    out_type = (
        jax.ShapeDtypeStruct((ACC_ROWS, D), jnp.float32),
        jax.ShapeDtypeStruct((ACC_ROWS,), jnp.float32),
        jax.ShapeDtypeStruct((ACC_ROWS, D), jnp.float32),
        jax.ShapeDtypeStruct((ACC_ROWS,), jnp.float32),
    )
    scratch = [
        pltpu.VMEM_SHARED((ACC_ROWS, D), jnp.float32),
        pltpu.VMEM_SHARED((ACC_ROWS,), jnp.float32),
        pltpu.VMEM((CHUNK,), jnp.int32),
        pltpu.VMEM((CHUNK,), jnp.int32),
        pltpu.VMEM((CHUNK,), jnp.int32),
        pltpu.VMEM((CHUNK,), jnp.int32),
        pltpu.VMEM((CHUNK, D), jnp.float32),
        pltpu.VMEM((CHUNK, D), jnp.float32),
        pltpu.VMEM((CHUNK,), jnp.float32),
        pltpu.SemaphoreType.DMA,
        pltpu.SemaphoreType.DMA,
        pltpu.SemaphoreType.DMA,
        pltpu.SemaphoreType.DMA,
    ]

    @functools.partial(pl.kernel, mesh=mesh, out_type=out_type,
                       scratch_types=scratch)
    def sc_kernel(x_hbm, src_hbm, dst_hbm, z2d_hbm, z1d_hbm,
                  sum1_hbm, deg1_hbm, sum2_hbm, deg2_hbm,
                  acc, deg, gi0, gi1, si0, si1, rows0, rows1, ones,
                  gsem0, gsem1, ssem0, ssem1):
        c = lax.axis_index("c")
        s = lax.axis_index("s")
        gi = (gi0, gi1)
        si = (si0, si1)
        rows = (rows0, rows1)
        gsem = (gsem0, gsem1)
        ssem = (ssem0, ssem1)

        for i in range(CHUNK // 16):
            ones[pl.ds(i * 16, 16)] = jnp.ones((16,), jnp.float32)

        pltpu.sync_copy(z2d_hbm, acc.at[pl.ds(s * ROWS_PER_TILE,
                                              ROWS_PER_TILE)])
        @pl.when(s == 0)
        def _():
            pltpu.sync_copy(z1d_hbm, deg)
        plsc.subcore_barrier()

        def run_direction(gidx_hbm, sidx_hbm):
            base = s * EDGES_PER_TILE

            pltpu.async_copy(z2d_hbm.at[pl.ds(0, CHUNK)], rows[0], ssem[0])
            pltpu.async_copy(z2d_hbm.at[pl.ds(0, CHUNK)], rows[1], ssem[1])

            def stage_front(k, h):
                off = base + k * CHUNK
                pltpu.sync_copy(gidx_hbm.at[pl.ds(off, CHUNK)], gi[h])
                pltpu.make_async_copy(rows[h], acc.at[si[h]], ssem[h]).wait()
                pltpu.async_copy(x_hbm.at[gi[h]], rows[h], gsem[h])
                pltpu.sync_copy(sidx_hbm.at[pl.ds(off, CHUNK)], si[h])
                pltpu.sync_copy(ones, deg.at[si[h]], add=True)

            def stage_back(h):
                pltpu.make_async_copy(x_hbm.at[gi[h]], rows[h],
                                      gsem[h]).wait()
                pltpu.async_copy(rows[h], acc.at[si[h]], ssem[h], add=True)

            def body(i, _):
                k0 = 2 * i
                stage_front(k0, 0)
                stage_front(k0 + 1, 1)
                stage_back(0)
                stage_back(1)
                return _

            lax.fori_loop(0, CHUNKS_PER_TILE // 2, body, None)
            pltpu.make_async_copy(rows[0], acc.at[si[0]], ssem[0]).wait()
            pltpu.make_async_copy(rows[1], acc.at[si[1]], ssem[1]).wait()

        @pl.when(c == 0)
        def _():
            run_direction(src_hbm, dst_hbm)

        @pl.when(c == 1)
        def _():
            run_direction(dst_hbm, src_hbm)

        plsc.subcore_barrier()

        r0 = s * ROWS_PER_TILE

        @pl.when(c == 0)
        def _():
            pltpu.sync_copy(acc.at[pl.ds(r0, ROWS_PER_TILE)],
                            sum1_hbm.at[pl.ds(r0, ROWS_PER_TILE)])
            @pl.when(s == 0)
            def _():
                pltpu.sync_copy(deg, deg1_hbm)

        @pl.when(c == 1)
        def _():
            pltpu.sync_copy(acc.at[pl.ds(r0, ROWS_PER_TILE)],
                            sum2_hbm.at[pl.ds(r0, ROWS_PER_TILE)])
            @pl.when(s == 0)
            def _():
                pltpu.sync_copy(deg, deg2_hbm)

    return sc_kernel(x_pad, src_pad, dst_pad, z2d, z1d)


BLK = 1000


def _tc_combine_body(x_ref, w1_ref, w2_ref, w3_ref, b_ref,
                     sum1_ref, deg1_ref, sum2_ref, deg2_ref, out_ref):
    f32 = jnp.float32
    dn = (((1,), (1,)), ((), ()))
    x = x_ref[...]
    m1 = sum1_ref[...] / jnp.maximum(deg1_ref[...], 1.0)
    m2 = sum2_ref[...] / jnp.maximum(deg2_ref[...], 1.0)
    out = lax.dot_general(x, w3_ref[...], dn, preferred_element_type=f32,
                          precision=lax.Precision.HIGHEST)
    out += b_ref[0, :][None, :]
    out += (1.0 - ALPHA) * lax.dot_general(m1, w1_ref[...], dn,
                                           preferred_element_type=f32,
                                           precision=lax.Precision.HIGHEST)
    out += ALPHA * lax.dot_general(m2, w2_ref[...], dn,
                                   preferred_element_type=f32,
                                   precision=lax.Precision.HIGHEST)
    out_ref[...] = out


def _tc_combine(x, W_s2d, W_d2s, W_self, b2d, sum1, deg1, sum2, deg2):
    n = x.shape[0]
    grid = (n // BLK,)
    full128 = pl.BlockSpec((128, 128), lambda i: (0, 0))
    rows = pl.BlockSpec((BLK, 128), lambda i: (i, 0))
    col = pl.BlockSpec((BLK, 1), lambda i: (i, 0))
    return pl.pallas_call(
        _tc_combine_body,
        grid=grid,
        in_specs=[rows, full128, full128, full128,
                  pl.BlockSpec((8, 128), lambda i: (0, 0)),
                  rows, col, rows, col],
        out_specs=rows,
        out_shape=jax.ShapeDtypeStruct((n, 128), jnp.float32),
    )(x, W_s2d, W_d2s, W_self, b2d, sum1, deg1, sum2, deg2)


def kernel(x, edge_index, W_s2d, W_d2s, W_self, b_self):
    ei = edge_index.astype(jnp.int32)
    pad_len = E_PAD - ei.shape[1]
    pad = jnp.full((pad_len,), N_NODES, jnp.int32)
    src_pad = jnp.concatenate([ei[0], pad])
    dst_pad = jnp.concatenate([ei[1], pad])
    x_pad = jnp.concatenate([x, jnp.zeros((X_PAD_ROWS - N_NODES, D), x.dtype)])
    z2d = jnp.zeros((ROWS_PER_TILE, D), jnp.float32)
    z1d = jnp.zeros((ACC_ROWS,), jnp.float32)

    sum1, deg1, sum2, deg2 = _sc_aggregate(x_pad, src_pad, dst_pad, z2d, z1d)

    b2d = jnp.broadcast_to(b_self[None, :], (8, D))
    return _tc_combine(x, W_s2d, W_d2s, W_self, b2d,
                       sum1[:N_NODES], deg1[:N_NODES, None],
                       sum2[:N_NODES], deg2[:N_NODES, None])


# speedup vs baseline: 1.1690x; 1.1690x over previous
import functools

import jax
import jax.numpy as jnp
from jax import lax
from jax.experimental import pallas as pl
from jax.experimental.pallas import tpu as pltpu
from jax.experimental.pallas import tpu_sc as plsc

N_NODES = 10000
D = 128
ALPHA = 0.5

NUM_CORES = 2
NUM_TILES = 16
CHUNK = 128
CHUNKS_PER_TILE = 157
EDGES_PER_TILE = CHUNKS_PER_TILE * CHUNK
E_PAD = EDGES_PER_TILE * NUM_TILES
ACC_ROWS = 10112
ROWS_PER_TILE = ACC_ROWS // NUM_TILES
X_PAD_ROWS = 10016


def _sc_aggregate(x_pad, src_pad, dst_pad, z2d, z1d):
    mesh = plsc.VectorSubcoreMesh(core_axis_name="c", subcore_axis_name="s")
    out_type = (
        jax.ShapeDtypeStruct((ACC_ROWS, D), jnp.float32),
        jax.ShapeDtypeStruct((ACC_ROWS,), jnp.float32),
        jax.ShapeDtypeStruct((ACC_ROWS, D), jnp.float32),
        jax.ShapeDtypeStruct((ACC_ROWS,), jnp.float32),
    )
    scratch = [
        pltpu.VMEM_SHARED((ACC_ROWS, D), jnp.float32),
        pltpu.VMEM_SHARED((ACC_ROWS,), jnp.float32),
        pltpu.VMEM((CHUNK,), jnp.int32),
        pltpu.VMEM((CHUNK,), jnp.int32),
        pltpu.VMEM((CHUNK, D), jnp.float32),
        pltpu.VMEM((CHUNK,), jnp.float32),
        pltpu.SemaphoreType.DMA,
        pltpu.SemaphoreType.DMA,
    ]

    @functools.partial(pl.kernel, mesh=mesh, out_type=out_type,
                       scratch_types=scratch)
    def sc_kernel(x_hbm, src_hbm, dst_hbm, z2d_hbm, z1d_hbm,
                  sum1_hbm, deg1_hbm, sum2_hbm, deg2_hbm,
                  acc, deg, idx_g, idx_s, rows, ones, sem, ssem):
        c = lax.axis_index("c")
        s = lax.axis_index("s")

        for i in range(CHUNK // 16):
            ones[pl.ds(i * 16, 16)] = jnp.ones((16,), jnp.float32)

        pltpu.sync_copy(z2d_hbm, acc.at[pl.ds(s * ROWS_PER_TILE,
                                              ROWS_PER_TILE)])
        @pl.when(s == 0)
        def _():
            pltpu.sync_copy(z1d_hbm, deg)
        plsc.subcore_barrier()

        def run_direction(gidx_hbm, sidx_hbm):
            base = s * EDGES_PER_TILE

            pltpu.async_copy(z2d_hbm.at[pl.ds(0, CHUNK)], rows, ssem)

            def body(k, _):
                off = base + k * CHUNK
                pltpu.sync_copy(gidx_hbm.at[pl.ds(off, CHUNK)], idx_g)
                pltpu.make_async_copy(rows, acc.at[idx_s], ssem).wait()
                gd = pltpu.async_copy(x_hbm.at[idx_g], rows, sem)
                pltpu.sync_copy(sidx_hbm.at[pl.ds(off, CHUNK)], idx_s)
                pltpu.sync_copy(ones, deg.at[idx_s], add=True)
                gd.wait()
                pltpu.async_copy(rows, acc.at[idx_s], ssem, add=True)
                return _

            lax.fori_loop(0, CHUNKS_PER_TILE, body, None)
            pltpu.make_async_copy(rows, acc.at[idx_s], ssem).wait()

        @pl.when(c == 0)
        def _():
            run_direction(src_hbm, dst_hbm)

        @pl.when(c == 1)
        def _():
            run_direction(dst_hbm, src_hbm)

        plsc.subcore_barrier()

        r0 = s * ROWS_PER_TILE

        @pl.when(c == 0)
        def _():
            pltpu.sync_copy(acc.at[pl.ds(r0, ROWS_PER_TILE)],
                            sum1_hbm.at[pl.ds(r0, ROWS_PER_TILE)])
            @pl.when(s == 0)
            def _():
                pltpu.sync_copy(deg, deg1_hbm)

        @pl.when(c == 1)
        def _():
            pltpu.sync_copy(acc.at[pl.ds(r0, ROWS_PER_TILE)],
                            sum2_hbm.at[pl.ds(r0, ROWS_PER_TILE)])
            @pl.when(s == 0)
            def _():
                pltpu.sync_copy(deg, deg2_hbm)

    return sc_kernel(x_pad, src_pad, dst_pad, z2d, z1d)


BLK = 1000


def _tc_combine_body(x_ref, w1_ref, w2_ref, w3_ref, b_ref,
                     sum1_ref, deg1_ref, sum2_ref, deg2_ref, out_ref):
    f32 = jnp.float32
    dn = (((1,), (1,)), ((), ()))
    x = x_ref[...]
    m1 = sum1_ref[...] / jnp.maximum(deg1_ref[...], 1.0)
    m2 = sum2_ref[...] / jnp.maximum(deg2_ref[...], 1.0)
    out = lax.dot_general(x, w3_ref[...], dn, preferred_element_type=f32,
                          precision=lax.Precision.HIGHEST)
    out += b_ref[0, :][None, :]
    out += (1.0 - ALPHA) * lax.dot_general(m1, w1_ref[...], dn,
                                           preferred_element_type=f32,
                                           precision=lax.Precision.HIGHEST)
    out += ALPHA * lax.dot_general(m2, w2_ref[...], dn,
                                   preferred_element_type=f32,
                                   precision=lax.Precision.HIGHEST)
    out_ref[...] = out


def _tc_combine(x, W_s2d, W_d2s, W_self, b2d, sum1, deg1, sum2, deg2):
    n = x.shape[0]
    grid = (n // BLK,)
    full128 = pl.BlockSpec((128, 128), lambda i: (0, 0))
    rows = pl.BlockSpec((BLK, 128), lambda i: (i, 0))
    col = pl.BlockSpec((BLK, 1), lambda i: (i, 0))
    return pl.pallas_call(
        _tc_combine_body,
        grid=grid,
        in_specs=[rows, full128, full128, full128,
                  pl.BlockSpec((8, 128), lambda i: (0, 0)),
                  rows, col, rows, col],
        out_specs=rows,
        out_shape=jax.ShapeDtypeStruct((n, 128), jnp.float32),
    )(x, W_s2d, W_d2s, W_self, b2d, sum1, deg1, sum2, deg2)


def kernel(x, edge_index, W_s2d, W_d2s, W_self, b_self):
    ei = edge_index.astype(jnp.int32)
    pad_len = E_PAD - ei.shape[1]
    pad = jnp.full((pad_len,), N_NODES, jnp.int32)
    src_pad = jnp.concatenate([ei[0], pad])
    dst_pad = jnp.concatenate([ei[1], pad])
    x_pad = jnp.concatenate([x, jnp.zeros((X_PAD_ROWS - N_NODES, D), x.dtype)])
    z2d = jnp.zeros((ROWS_PER_TILE, D), jnp.float32)
    z1d = jnp.zeros((ACC_ROWS,), jnp.float32)

    sum1, deg1, sum2, deg2 = _sc_aggregate(x_pad, src_pad, dst_pad, z2d, z1d)

    b2d = jnp.broadcast_to(b_self[None, :], (8, D))
    return _tc_combine(x, W_s2d, W_d2s, W_self, b2d,
                       sum1, deg1[:, None], sum2, deg2[:, None])
